# Initial kernel scaffold; baseline (speedup 1.0000x reference)
import jax
import jax.numpy as jnp
from jax.experimental import pallas as pl


def kernel(x_nchw, w_oihw, bias, gamma, beta):
    raise NotImplementedError("write your pallas kernel here")


# speedup vs baseline: 2.2120x; 2.2120x over previous
import jax
import jax.numpy as jnp
from jax import lax
from jax.experimental import pallas as pl
from jax.experimental.pallas import tpu as pltpu

_EPS = 1e-5


def _roll_lanes(x, k, size):
    k %= size
    if k == 0:
        return x
    return jnp.concatenate([x[:, k:], x[:, :k]], axis=1)


def _conv_stats_kernel(h_w, x_ref, w_ref, b_ref, conv_ref, sum_ref, ssq_ref):
    H, W = h_w
    x = x_ref[0].astype(jnp.bfloat16)
    C, HW = x.shape

    lane = lax.broadcasted_iota(jnp.int32, (1, HW), 1)
    wid = lane % W
    hid = lane // W
    w_ok = [wid >= 1, None, wid <= W - 2]
    h_ok = [hid >= 1, None, hid <= H - 2]

    cols = []
    for ky in range(3):
        for kx in range(3):
            off = (ky - 1) * W + (kx - 1)
            col = _roll_lanes(x, off, HW)
            m = h_ok[ky] if w_ok[kx] is None else (
                w_ok[kx] if h_ok[ky] is None else jnp.logical_and(h_ok[ky], w_ok[kx]))
            if m is not None:
                col = jnp.where(m, col, jnp.bfloat16(0))
            cols.append(col)
    patches = jnp.concatenate(cols, axis=0)

    acc = jnp.dot(w_ref[...], patches, preferred_element_type=jnp.float32)
    acc = acc + b_ref[...]

    conv_ref[0] = acc.astype(conv_ref.dtype)
    sum_ref[0] = jnp.sum(acc, axis=1, keepdims=True)
    ssq_ref[0] = jnp.sum(acc * acc, axis=1, keepdims=True)


def _bn_relu_kernel(conv_ref, scale_ref, shift_ref, o_ref):
    y = conv_ref[0].astype(jnp.float32) * scale_ref[...] + shift_ref[...]
    o_ref[0] = jnp.maximum(y, 0.0)


def kernel(x_nchw, w_oihw, bias, gamma, beta):
    N, C, H, W = x_nchw.shape
    HW = H * W
    x = x_nchw.reshape(N, C, HW)

    w_lhs = jnp.transpose(w_oihw, (0, 2, 3, 1)).reshape(C, 9 * C)
    w_lhs = w_lhs.astype(jnp.bfloat16)
    b_col = bias.reshape(C, 1).astype(jnp.float32)

    flops = 2 * N * HW * (9 * C) * C
    bytes_accessed = (N * C * HW) * (4 + 2) + (9 * C * C) * 2 + 2 * N * C * 4

    conv, psum, pssq = pl.pallas_call(
        lambda *refs: _conv_stats_kernel((H, W), *refs),
        out_shape=(
            jax.ShapeDtypeStruct((N, C, HW), jnp.bfloat16),
            jax.ShapeDtypeStruct((N, C, 1), jnp.float32),
            jax.ShapeDtypeStruct((N, C, 1), jnp.float32),
        ),
        grid=(N,),
        in_specs=[
            pl.BlockSpec((1, C, HW), lambda n: (n, 0, 0)),
            pl.BlockSpec((C, 9 * C), lambda n: (0, 0)),
            pl.BlockSpec((C, 1), lambda n: (0, 0)),
        ],
        out_specs=(
            pl.BlockSpec((1, C, HW), lambda n: (n, 0, 0)),
            pl.BlockSpec((1, C, 1), lambda n: (n, 0, 0)),
            pl.BlockSpec((1, C, 1), lambda n: (n, 0, 0)),
        ),
        compiler_params=pltpu.CompilerParams(dimension_semantics=("parallel",)),
        cost_estimate=pl.CostEstimate(
            flops=flops, transcendentals=0, bytes_accessed=bytes_accessed),
    )(x, w_lhs, b_col)

    m_total = float(N * HW)
    ch_sum = jnp.sum(psum, axis=0)
    ch_ssq = jnp.sum(pssq, axis=0)
    mean = ch_sum / m_total
    var = jnp.maximum(ch_ssq / m_total - mean * mean, 0.0)
    inv = lax.rsqrt(var + _EPS)
    scale = gamma.reshape(C, 1).astype(jnp.float32) * inv
    shift = beta.reshape(C, 1).astype(jnp.float32) - mean * scale

    y = pl.pallas_call(
        _bn_relu_kernel,
        out_shape=jax.ShapeDtypeStruct((N, C, HW), jnp.float32),
        grid=(N,),
        in_specs=[
            pl.BlockSpec((1, C, HW), lambda n: (n, 0, 0)),
            pl.BlockSpec((C, 1), lambda n: (0, 0)),
            pl.BlockSpec((C, 1), lambda n: (0, 0)),
        ],
        out_specs=pl.BlockSpec((1, C, HW), lambda n: (n, 0, 0)),
        compiler_params=pltpu.CompilerParams(dimension_semantics=("parallel",)),
    )(conv, scale, shift)

    return y.reshape(N, C, H, W)
